# baseline (speedup 1.0000x reference)
import functools

import jax
import jax.numpy as jnp
from jax import lax
from jax.experimental import pallas as pl
from jax.experimental.pallas import tpu as pltpu
from jax.experimental.pallas import tpu_sc as plsc

NC = 2
NS = 16
NW = NC * NS
K = 96


def _mesh():
    return plsc.VectorSubcoreMesh(core_axis_name="c", subcore_axis_name="s")


def _fill(buf, rows, d, val):
    v = jnp.full((16,), val, jnp.float32)
    for r in range(rows):
        for j in range(d // 16):
            buf[r, pl.ds(j * 16, 16)] = v


def _zero_acc_slice(zbuf, acc, base, rpt):
    nfull = rpt // K
    rem = rpt - nfull * K
    for j in range(nfull):
        pltpu.sync_copy(zbuf, acc.at[pl.ds(base + j * K, K)])
    if rem:
        pltpu.sync_copy(zbuf.at[pl.ds(0, rem)],
                        acc.at[pl.ds(base + nfull * K, rem)])


def _make_deg_kernel(np_, ch):
    rpt = np_ // NS

    @functools.partial(
        pl.kernel,
        out_type=jax.ShapeDtypeStruct((NC, np_, 16), jnp.float32),
        mesh=_mesh(),
        compiler_params=pltpu.CompilerParams(use_tc_tiling_on_sc=False),
        scratch_types=[
            pltpu.VMEM((ch + 1, K), jnp.int32),
            pltpu.VMEM((K, 16), jnp.float32),
            pltpu.VMEM_SHARED((np_, 16), jnp.float32),
        ],
    )
    def deg_kernel(dst3, out, dstv, ones, acc):
        c = lax.axis_index("c")
        s = lax.axis_index("s")
        wid = c * NS + s
        _fill(ones, K, 16, 0.0)
        _zero_acc_slice(ones, acc, s * rpt, rpt)
        _fill(ones, K, 16, 1.0)
        pltpu.sync_copy(dst3.at[wid], dstv)
        plsc.subcore_barrier()

        def body(g, carry):
            pltpu.sync_copy(ones, acc.at[dstv.at[g]], add=True)
            return carry

        lax.fori_loop(0, ch, body, 0)
        plsc.subcore_barrier()
        pltpu.sync_copy(acc.at[pl.ds(s * rpt, rpt)],
                        out.at[c, pl.ds(s * rpt, rpt)])

    return deg_kernel


def _make_agg_kernel(np_, d, ch):
    rpt = np_ // NS
    cp = pltpu.CompilerParams(use_tc_tiling_on_sc=False)

    @functools.partial(
        pl.kernel,
        out_type=jax.ShapeDtypeStruct((NC, np_, d), jnp.float32),
        mesh=_mesh(),
        compiler_params=cp,
        scratch_types=[
            pltpu.VMEM((ch + 1, K), jnp.int32),
            pltpu.VMEM((ch + 1, K), jnp.int32),
            pltpu.VMEM((K, d), jnp.float32),
            pltpu.VMEM((K, d), jnp.float32),
            pltpu.VMEM_SHARED((np_, d), jnp.float32),
            pltpu.SemaphoreType.DMA,
            pltpu.SemaphoreType.DMA,
        ],
    )
    def agg_kernel(hw2, src3, dst3, out, srcv, dstv, rows0, rows1, acc,
                   sem0, sem1):
        c = lax.axis_index("c")
        s = lax.axis_index("s")
        wid = c * NS + s

        _fill(rows0, K, d, 0.0)

        @pl.when(c == 0)
        def _():
            pltpu.sync_copy(hw2.at[pl.ds(s * rpt, rpt)],
                            acc.at[pl.ds(s * rpt, rpt)])

        @pl.when(c == 1)
        def _():
            _zero_acc_slice(rows0, acc, s * rpt, rpt)

        pltpu.sync_copy(src3.at[wid], srcv)
        pltpu.sync_copy(dst3.at[wid], dstv)
        plsc.subcore_barrier()

        pltpu.async_copy(hw2.at[srcv.at[0]], rows0, sem0)

        def body(g2, carry):
            g = g2 * 2
            pltpu.async_copy(hw2.at[srcv.at[g + 1]], rows1, sem1)
            pltpu.make_async_copy(hw2.at[srcv.at[g]], rows0, sem0).wait()
            pltpu.sync_copy(rows0, acc.at[dstv.at[g]], add=True)
            pltpu.async_copy(hw2.at[srcv.at[g + 2]], rows0, sem0)
            pltpu.make_async_copy(hw2.at[srcv.at[g + 1]], rows1, sem1).wait()
            pltpu.sync_copy(rows1, acc.at[dstv.at[g + 1]], add=True)
            return carry

        lax.fori_loop(0, ch // 2, body, 0)
        pltpu.make_async_copy(hw2.at[srcv.at[0]], rows0, sem0).wait()

        plsc.subcore_barrier()
        pltpu.sync_copy(acc.at[pl.ds(s * rpt, rpt)],
                        out.at[c, pl.ds(s * rpt, rpt)])

    return agg_kernel


def _mm1_body(x_ref, w_ref, degp_ref, hw2_ref, dinv_ref):
    deg = degp_ref[0, :, 0:1] + degp_ref[1, :, 0:1] + 1.0
    dinv = lax.rsqrt(deg)
    dinv_ref[...] = dinv
    hw2_ref[...] = jnp.dot(x_ref[...], w_ref[...],
                           preferred_element_type=jnp.float32) * dinv


def _mm_body(p_ref, dinv_ref, b_ref, w_ref, act_ref, hw2_ref, *, relu):
    dinv = dinv_ref[...]
    act = (p_ref[0] + p_ref[1]) * dinv + b_ref[...]
    if relu:
        act = jnp.maximum(act, 0.0)
    act_ref[...] = act
    hw2_ref[...] = jnp.dot(act, w_ref[...],
                           preferred_element_type=jnp.float32) * dinv


def _ep_body(p_ref, dinv_ref, b_ref, out_ref):
    out_ref[...] = (p_ref[0] + p_ref[1]) * dinv_ref[...] + b_ref[...]


def kernel(x, edge_index, W1, b1, W2, b2, Wc, bc):
    n, in_dim = x.shape
    hid = W1.shape[1]
    nc = Wc.shape[1]
    e = edge_index.shape[1]

    np_ = -(-n // 128) * 128
    src = edge_index[0]
    dst = edge_index[1]
    ch = 2 * (-(-e // (NW * K * 2)))
    pad = NW * K * ch - e
    src3 = jnp.concatenate(
        [jnp.concatenate([src, jnp.zeros((pad,), jnp.int32)]).reshape(NW, ch, K),
         jnp.zeros((NW, 1, K), jnp.int32)], axis=1)
    dst3 = jnp.concatenate(
        [jnp.concatenate([dst, jnp.full((pad,), n, jnp.int32)]).reshape(NW, ch, K),
         jnp.full((NW, 1, K), n, jnp.int32)], axis=1)

    deg_p = _make_deg_kernel(np_, ch)(dst3)

    R = np_ // 8
    grid = (np_ // R,)

    mm1 = pl.pallas_call(
        _mm1_body,
        grid=grid,
        in_specs=[
            pl.BlockSpec((R, in_dim), lambda i: (i, 0)),
            pl.BlockSpec((in_dim, hid), lambda i: (0, 0)),
            pl.BlockSpec((NC, R, 16), lambda i: (0, i, 0)),
        ],
        out_specs=[
            pl.BlockSpec((R, hid), lambda i: (i, 0)),
            pl.BlockSpec((R, 1), lambda i: (i, 0)),
        ],
        out_shape=[
            jax.ShapeDtypeStruct((np_, hid), jnp.float32),
            jax.ShapeDtypeStruct((np_, 1), jnp.float32),
        ],
    )
    hw2_1, dinv = mm1(x, W1, deg_p)

    agg_h = _make_agg_kernel(np_, hid, ch)
    p1 = agg_h(hw2_1, src3, dst3)

    def mm_call(p, b, w, d_out, relu):
        return pl.pallas_call(
            functools.partial(_mm_body, relu=relu),
            grid=grid,
            in_specs=[
                pl.BlockSpec((NC, R, p.shape[2]), lambda i: (0, i, 0)),
                pl.BlockSpec((R, 1), lambda i: (i, 0)),
                pl.BlockSpec((1, b.shape[1]), lambda i: (0, 0)),
                pl.BlockSpec((w.shape[0], d_out), lambda i: (0, 0)),
            ],
            out_specs=[
                pl.BlockSpec((R, b.shape[1]), lambda i: (i, 0)),
                pl.BlockSpec((R, d_out), lambda i: (i, 0)),
            ],
            out_shape=[
                jax.ShapeDtypeStruct((n, b.shape[1]), jnp.float32),
                jax.ShapeDtypeStruct((np_, d_out), jnp.float32),
            ],
        )(p, dinv, b, w)

    h1, hw2_2 = mm_call(p1, b1.reshape(1, hid), W2, hid, relu=True)
    p2 = agg_h(hw2_2, src3, dst3)

    h2, hw2_3 = mm_call(p2, b2.reshape(1, hid), Wc, nc, relu=False)
    p3 = _make_agg_kernel(np_, nc, ch)(hw2_3, src3, dst3)

    out = pl.pallas_call(
        _ep_body,
        grid=grid,
        in_specs=[
            pl.BlockSpec((NC, R, nc), lambda i: (0, i, 0)),
            pl.BlockSpec((R, 1), lambda i: (i, 0)),
            pl.BlockSpec((1, nc), lambda i: (0, 0)),
        ],
        out_specs=pl.BlockSpec((R, nc), lambda i: (i, 0)),
        out_shape=jax.ShapeDtypeStruct((n, nc), jnp.float32),
    )(p3, dinv, bc.reshape(1, nc))

    return (out, h1, h2)


# speedup vs baseline: 1.3347x; 1.3347x over previous
import functools

import jax
import jax.numpy as jnp
from jax import lax
from jax.experimental import pallas as pl
from jax.experimental.pallas import tpu as pltpu
from jax.experimental.pallas import tpu_sc as plsc

NC = 2
NS = 16
NW = NC * NS
K = 128


def _mesh():
    return plsc.VectorSubcoreMesh(core_axis_name="c", subcore_axis_name="s")


def _fill(buf, rows, d, val):
    v = jnp.full((16,), val, jnp.float32)
    for r in range(rows):
        for j in range(d // 16):
            buf[r, pl.ds(j * 16, 16)] = v


def _zero_acc_slice(zbuf, acc, base, rpt):
    nfull = rpt // K
    rem = rpt - nfull * K
    for j in range(nfull):
        pltpu.sync_copy(zbuf, acc.at[pl.ds(base + j * K, K)])
    if rem:
        pltpu.sync_copy(zbuf.at[pl.ds(0, rem)],
                        acc.at[pl.ds(base + nfull * K, rem)])


def _make_deg_kernel(np_, ch):
    rpt = np_ // NS

    @functools.partial(
        pl.kernel,
        out_type=jax.ShapeDtypeStruct((NC, np_, 16), jnp.float32),
        mesh=_mesh(),
        compiler_params=pltpu.CompilerParams(use_tc_tiling_on_sc=False),
        scratch_types=[
            pltpu.VMEM((ch, K), jnp.int32),
            pltpu.VMEM((K, 16), jnp.float32),
            pltpu.VMEM_SHARED((np_, 16), jnp.float32),
        ],
    )
    def deg_kernel(dst3, out, dstv, ones, acc):
        c = lax.axis_index("c")
        s = lax.axis_index("s")
        wid = c * NS + s
        _fill(ones, K, 16, 0.0)
        _zero_acc_slice(ones, acc, s * rpt, rpt)
        _fill(ones, K, 16, 1.0)
        pltpu.sync_copy(dst3.at[wid], dstv)
        plsc.subcore_barrier()

        def body(g, carry):
            pltpu.sync_copy(ones, acc.at[dstv.at[g]], add=True)
            return carry

        lax.fori_loop(0, ch, body, 0)
        plsc.subcore_barrier()
        pltpu.sync_copy(acc.at[pl.ds(s * rpt, rpt)],
                        out.at[c, pl.ds(s * rpt, rpt)])

    return deg_kernel


def _make_agg_kernel(np_, d, ch):
    rpt = np_ // NS
    cp = (pltpu.CompilerParams(use_tc_tiling_on_sc=False)
          if d < 128 else None)

    @functools.partial(
        pl.kernel,
        out_type=jax.ShapeDtypeStruct((NC, np_, d), jnp.float32),
        mesh=_mesh(),
        compiler_params=cp,
        scratch_types=[
            pltpu.VMEM((ch, K), jnp.int32),
            pltpu.VMEM((ch, K), jnp.int32),
            pltpu.VMEM((K, d), jnp.float32),
            pltpu.VMEM_SHARED((np_, d), jnp.float32),
            pltpu.SemaphoreType.DMA,
        ],
    )
    def agg_kernel(hw2, src3, dst3, out, srcv, dstv, rows0, acc, sem0):
        c = lax.axis_index("c")
        s = lax.axis_index("s")
        wid = c * NS + s

        _fill(rows0, K, d, 0.0)

        @pl.when(c == 0)
        def _():
            pltpu.sync_copy(hw2.at[pl.ds(s * rpt, rpt)],
                            acc.at[pl.ds(s * rpt, rpt)])

        @pl.when(c == 1)
        def _():
            _zero_acc_slice(rows0, acc, s * rpt, rpt)

        pltpu.sync_copy(src3.at[wid], srcv)
        pltpu.sync_copy(dst3.at[wid], dstv)
        plsc.subcore_barrier()

        def body(g, carry):
            pltpu.async_copy(hw2.at[srcv.at[g]], rows0, sem0).wait()
            pltpu.sync_copy(rows0, acc.at[dstv.at[g]], add=True)
            return carry

        lax.fori_loop(0, ch, body, 0)
        plsc.subcore_barrier()
        pltpu.sync_copy(acc.at[pl.ds(s * rpt, rpt)],
                        out.at[c, pl.ds(s * rpt, rpt)])

    return agg_kernel


def _mm1_body(x_ref, w_ref, degp_ref, hw2_ref, dinv_ref):
    deg = degp_ref[0, :, 0:1] + degp_ref[1, :, 0:1] + 1.0
    dinv = lax.rsqrt(deg)
    dinv_ref[...] = dinv
    hw2_ref[...] = jnp.dot(x_ref[...], w_ref[...],
                           preferred_element_type=jnp.float32) * dinv


def _mm_body(p_ref, dinv_ref, b_ref, w_ref, act_ref, hw2_ref, *, relu):
    dinv = dinv_ref[...]
    act = (p_ref[0] + p_ref[1]) * dinv + b_ref[...]
    if relu:
        act = jnp.maximum(act, 0.0)
    act_ref[...] = act
    hw2_ref[...] = jnp.dot(act, w_ref[...],
                           preferred_element_type=jnp.float32) * dinv


def _ep_body(p_ref, dinv_ref, b_ref, out_ref):
    out_ref[...] = (p_ref[0] + p_ref[1]) * dinv_ref[...] + b_ref[...]


def kernel(x, edge_index, W1, b1, W2, b2, Wc, bc):
    n, in_dim = x.shape
    hid = W1.shape[1]
    nc = Wc.shape[1]
    e = edge_index.shape[1]

    np_ = -(-n // 128) * 128
    src = edge_index[0]
    dst = edge_index[1]
    ch = -(-e // (NW * K))
    pad = NW * K * ch - e
    src3 = jnp.concatenate([src, jnp.zeros((pad,), jnp.int32)]).reshape(NW, ch, K)
    dst3 = jnp.concatenate([dst, jnp.full((pad,), n, jnp.int32)]).reshape(NW, ch, K)

    deg_p = _make_deg_kernel(np_, ch)(dst3)

    R = np_ // 8
    grid = (np_ // R,)

    mm1 = pl.pallas_call(
        _mm1_body,
        grid=grid,
        in_specs=[
            pl.BlockSpec((R, in_dim), lambda i: (i, 0)),
            pl.BlockSpec((in_dim, hid), lambda i: (0, 0)),
            pl.BlockSpec((NC, R, 16), lambda i: (0, i, 0)),
        ],
        out_specs=[
            pl.BlockSpec((R, hid), lambda i: (i, 0)),
            pl.BlockSpec((R, 1), lambda i: (i, 0)),
        ],
        out_shape=[
            jax.ShapeDtypeStruct((np_, hid), jnp.float32),
            jax.ShapeDtypeStruct((np_, 1), jnp.float32),
        ],
    )
    hw2_1, dinv = mm1(x, W1, deg_p)

    agg_h = _make_agg_kernel(np_, hid, ch)
    p1 = agg_h(hw2_1, src3, dst3)

    def mm_call(p, b, w, d_out, relu):
        return pl.pallas_call(
            functools.partial(_mm_body, relu=relu),
            grid=grid,
            in_specs=[
                pl.BlockSpec((NC, R, p.shape[2]), lambda i: (0, i, 0)),
                pl.BlockSpec((R, 1), lambda i: (i, 0)),
                pl.BlockSpec((1, b.shape[1]), lambda i: (0, 0)),
                pl.BlockSpec((w.shape[0], d_out), lambda i: (0, 0)),
            ],
            out_specs=[
                pl.BlockSpec((R, b.shape[1]), lambda i: (i, 0)),
                pl.BlockSpec((R, d_out), lambda i: (i, 0)),
            ],
            out_shape=[
                jax.ShapeDtypeStruct((n, b.shape[1]), jnp.float32),
                jax.ShapeDtypeStruct((np_, d_out), jnp.float32),
            ],
        )(p, dinv, b, w)

    h1, hw2_2 = mm_call(p1, b1.reshape(1, hid), W2, hid, relu=True)
    p2 = agg_h(hw2_2, src3, dst3)

    h2, hw2_3 = mm_call(p2, b2.reshape(1, hid), Wc, nc, relu=False)
    p3 = _make_agg_kernel(np_, nc, ch)(hw2_3, src3, dst3)

    out = pl.pallas_call(
        _ep_body,
        grid=grid,
        in_specs=[
            pl.BlockSpec((NC, R, nc), lambda i: (0, i, 0)),
            pl.BlockSpec((R, 1), lambda i: (i, 0)),
            pl.BlockSpec((1, nc), lambda i: (0, 0)),
        ],
        out_specs=pl.BlockSpec((R, nc), lambda i: (i, 0)),
        out_shape=jax.ShapeDtypeStruct((n, nc), jnp.float32),
    )(p3, dinv, bc.reshape(1, nc))

    return (out, h1, h2)
